# baseline (speedup 1.0000x reference)
import functools

import jax
import jax.numpy as jnp
from jax import lax
from jax.experimental import pallas as pl
from jax.experimental.pallas import tpu as pltpu
from jax.experimental.pallas import tpu_sc as plsc

_F32 = jnp.float32
_BN_EPS = 1e-5


def _pick_br(r):
    if r % 1000 == 0:
        return 1000
    return r


def _mm(xs, wts, bias=None, *, relu=False, adds=(), post_add=None,
        stats=False, bn=None, eps=None, pre_add=None, precision=None):
    xs = list(xs)
    wts = list(wts)
    adds = list(adds)
    R = xs[0].shape[0]
    N = wts[0].shape[1]
    BR = _pick_br(R)
    grid = R // BR
    nx = len(xs)
    na = len(adds)
    has_b = bias is not None
    has_pa = post_add is not None
    has_eps = eps is not None
    has_bn = bn is not None
    has_pre = pre_add is not None

    in_specs = []
    args = []
    for xa in xs:
        in_specs.append(pl.BlockSpec((BR, xa.shape[1]), lambda i: (i, 0)))
        args.append(xa)
    if has_pre:
        in_specs.append(pl.BlockSpec((BR, xs[0].shape[1]), lambda i: (i, 0)))
        args.append(pre_add)
    for wt in wts:
        in_specs.append(pl.BlockSpec(wt.shape, lambda i: (0, 0)))
        args.append(wt)
    if has_bn:
        st_in, g_in, bt_in, cnt = bn
        K0 = xs[0].shape[1]
        in_specs.append(pl.BlockSpec((8, K0), lambda i: (0, 0)))
        args.append(st_in)
        in_specs.append(pl.BlockSpec((1, K0), lambda i: (0, 0)))
        args.append(g_in.reshape(1, -1))
        in_specs.append(pl.BlockSpec((1, K0), lambda i: (0, 0)))
        args.append(bt_in.reshape(1, -1))
    if has_b:
        in_specs.append(pl.BlockSpec((1, N), lambda i: (0, 0)))
        args.append(bias.reshape(1, -1))
    for a in adds:
        in_specs.append(pl.BlockSpec((BR, N), lambda i: (i, 0)))
        args.append(a)
    if has_pa:
        in_specs.append(pl.BlockSpec((BR, N), lambda i: (i, 0)))
        args.append(post_add)
    if has_eps:
        in_specs.append(pl.BlockSpec((1, 1), lambda i: (0, 0)))
        args.append(jnp.reshape(eps, (1, 1)).astype(_F32))

    out_shape = [jax.ShapeDtypeStruct((R, N), _F32)]
    out_specs = [pl.BlockSpec((BR, N), lambda i: (i, 0))]
    if stats:
        out_shape.append(jax.ShapeDtypeStruct((8, N), _F32))
        out_specs.append(pl.BlockSpec((8, N), lambda i: (0, 0)))

    def body(*refs):
        p = 0
        xrefs = refs[p:p + nx]; p += nx
        if has_pre:
            pre_ref = refs[p]; p += 1
        wrefs = refs[p:p + nx]; p += nx
        if has_bn:
            st_ref, g_ref, bt_ref = refs[p:p + 3]; p += 3
        if has_b:
            b_ref = refs[p]; p += 1
        arefs = refs[p:p + na]; p += na
        if has_pa:
            pa_ref = refs[p]; p += 1
        if has_eps:
            eps_ref = refs[p]; p += 1
        out_ref = refs[p]; p += 1
        if stats:
            sto_ref = refs[p]; p += 1

        i = pl.program_id(0)
        if has_bn:
            stv = st_ref[...]
            s1 = stv[0:1, :]
            s2 = stv[1:2, :]
            mean = s1 / cnt
            var = s2 / cnt - mean * mean
            scale = g_ref[...] * lax.rsqrt(var + _BN_EPS)
            shift = bt_ref[...] - mean * scale
            x0 = jnp.maximum(xrefs[0][...] * scale + shift, 0.0)
        else:
            x0 = xrefs[0][...]
        if has_eps:
            x0 = x0 * (1.0 + eps_ref[0, 0])
        if has_pre:
            x0 = x0 + pre_ref[...]
        acc = jnp.dot(x0, wrefs[0][...], preferred_element_type=_F32,
                      precision=precision)
        for t in range(1, nx):
            acc = acc + jnp.dot(xrefs[t][...], wrefs[t][...],
                                preferred_element_type=_F32,
                                precision=precision)
        if has_b:
            acc = acc + b_ref[...]
        for ar in arefs:
            acc = acc + ar[...].astype(_F32)
        if relu:
            acc = jnp.maximum(acc, 0.0)
        if has_pa:
            acc = acc + pa_ref[...]
        out_ref[...] = acc
        if stats:
            c1 = jnp.sum(acc, axis=0, keepdims=True)
            c2 = jnp.sum(acc * acc, axis=0, keepdims=True)
            blk = jnp.concatenate(
                [c1, c2, jnp.zeros((6, N), _F32)], axis=0)

            @pl.when(i == 0)
            def _():
                sto_ref[...] = blk

            @pl.when(i != 0)
            def _():
                sto_ref[...] = sto_ref[...] + blk

    res = pl.pallas_call(
        body, grid=(grid,), in_specs=in_specs, out_specs=out_specs,
        out_shape=out_shape)(*args)
    if stats:
        return res[0], res[1]
    return res[0]


def _bn_relu(z, st, g, bt, cnt, relu=True):
    R, K = z.shape
    BR = _pick_br(R)
    grid = R // BR

    def body(z_ref, st_ref, g_ref, bt_ref, o_ref):
        stv = st_ref[...]
        mean = stv[0:1, :] / cnt
        var = stv[1:2, :] / cnt - mean * mean
        scale = g_ref[...] * lax.rsqrt(var + _BN_EPS)
        shift = bt_ref[...] - mean * scale
        o = z_ref[...] * scale + shift
        if relu:
            o = jnp.maximum(o, 0.0)
        o_ref[...] = o

    return pl.pallas_call(
        body, grid=(grid,),
        in_specs=[pl.BlockSpec((BR, K), lambda i: (i, 0)),
                  pl.BlockSpec((8, K), lambda i: (0, 0)),
                  pl.BlockSpec((1, K), lambda i: (0, 0)),
                  pl.BlockSpec((1, K), lambda i: (0, 0))],
        out_specs=pl.BlockSpec((BR, K), lambda i: (i, 0)),
        out_shape=jax.ShapeDtypeStruct((R, K), _F32),
    )(z, st, g.reshape(1, -1), bt.reshape(1, -1))


def _pool(hcat, batch, n_graphs):
    R, D = hcat.shape
    BRn = 2000 if R % 2000 == 0 else R
    grid = R // BRn
    b3 = batch.reshape(grid, 1, BRn)

    def body(h_ref, b_ref, o_ref):
        i = pl.program_id(0)
        bv = b_ref[...].reshape(1, BRn)
        gid = lax.broadcasted_iota(jnp.int32, (n_graphs, BRn), 0)
        oh = (gid == bv).astype(_F32)
        acc = jnp.dot(oh, h_ref[...], preferred_element_type=_F32,
                      precision=lax.Precision.HIGHEST)

        @pl.when(i == 0)
        def _():
            o_ref[...] = acc

        @pl.when(i != 0)
        def _():
            o_ref[...] = o_ref[...] + acc

    return pl.pallas_call(
        body, grid=(grid,),
        in_specs=[pl.BlockSpec((BRn, D), lambda i: (i, 0)),
                  pl.BlockSpec((1, 1, BRn), lambda i: (i, 0, 0))],
        out_specs=pl.BlockSpec((n_graphs, D), lambda i: (0, 0)),
        out_shape=jax.ShapeDtypeStruct((n_graphs, D), _F32),
    )(hcat, b3)


def _ff_fused(x, p):
    R, D = x.shape
    BR = 1000 if R % 1000 == 0 else R
    grid = R // BR
    w1t, w2t, w3t, wsct = p['W1'].T, p['W2'].T, p['W3'].T, p['Wsc'].T
    b1 = p['b1'].reshape(1, -1)
    b2 = p['b2'].reshape(1, -1)
    b3 = p['b3'].reshape(1, -1)
    bsc = p['bsc'].reshape(1, -1)

    def body(x_ref, w1_ref, w2_ref, w3_ref, wsc_ref,
             b1_ref, b2_ref, b3_ref, bsc_ref, o_ref):
        hi = None
        xv = x_ref[...]
        h1 = jnp.maximum(jnp.dot(xv, w1_ref[...], preferred_element_type=_F32,
                                 precision=hi) + b1_ref[...], 0.0)
        h2 = jnp.maximum(jnp.dot(h1, w2_ref[...], preferred_element_type=_F32,
                                 precision=hi) + b2_ref[...], 0.0)
        h3 = jnp.maximum(jnp.dot(h2, w3_ref[...], preferred_element_type=_F32,
                                 precision=hi) + b3_ref[...], 0.0)
        sc = jnp.dot(xv, wsc_ref[...], preferred_element_type=_F32,
                     precision=hi) + bsc_ref[...]
        o_ref[...] = h3 + sc

    wspec = pl.BlockSpec((D, D), lambda i: (0, 0))
    bspec = pl.BlockSpec((1, D), lambda i: (0, 0))
    return pl.pallas_call(
        body, grid=(grid,),
        in_specs=[pl.BlockSpec((BR, D), lambda i: (i, 0)),
                  wspec, wspec, wspec, wspec, bspec, bspec, bspec, bspec],
        out_specs=pl.BlockSpec((BR, D), lambda i: (i, 0)),
        out_shape=jax.ShapeDtypeStruct((R, D), _F32),
    )(x, w1t, w2t, w3t, wsct, b1, b2, b3, bsc)


def _edge_mms(ghs, ghd, e, wcx_t, wce_t, w1r_t, w1c_t, w1e_t, bc, be1):
    E, in_n = ghs.shape
    in_e = e.shape[1]
    dm = w1r_t.shape[1]
    BR = _pick_br(E)
    grid = E // BR

    def body(ghs_ref, ghd_ref, e_ref, wcx_ref, wce_ref, w1r_ref, w1c_ref,
             w1e_ref, bc_ref, be1_ref, c_ref, z1_ref, st_ref):
        i = pl.program_id(0)
        ghs_v = ghs_ref[...]
        e_v = e_ref[...]
        c = jnp.dot(ghs_v, wcx_ref[...], preferred_element_type=_F32)
        c = c + jnp.dot(e_v, wce_ref[...], preferred_element_type=_F32)
        c_ref[...] = jnp.maximum(c + bc_ref[...], 0.0)
        z1 = jnp.dot(ghs_v, w1r_ref[...], preferred_element_type=_F32)
        z1 = z1 + jnp.dot(ghd_ref[...], w1c_ref[...], preferred_element_type=_F32)
        z1 = z1 + jnp.dot(e_v, w1e_ref[...], preferred_element_type=_F32)
        z1 = z1 + be1_ref[...]
        z1_ref[...] = z1
        c1 = jnp.sum(z1, axis=0, keepdims=True)
        c2 = jnp.sum(z1 * z1, axis=0, keepdims=True)
        blk = jnp.concatenate([c1, c2, jnp.zeros((6, dm), _F32)], axis=0)

        @pl.when(i == 0)
        def _():
            st_ref[...] = blk

        @pl.when(i != 0)
        def _():
            st_ref[...] = st_ref[...] + blk

    row = lambda d: pl.BlockSpec((BR, d), lambda i: (i, 0))
    full = lambda sh: pl.BlockSpec(sh, lambda i: (0, 0))
    return pl.pallas_call(
        body, grid=(grid,),
        in_specs=[row(in_n), row(in_n), row(in_e),
                  full(wcx_t.shape), full(wce_t.shape), full(w1r_t.shape),
                  full(w1c_t.shape), full(w1e_t.shape),
                  full((1, in_n)), full((1, dm))],
        out_specs=[row(in_n), row(dm), full((8, dm))],
        out_shape=[jax.ShapeDtypeStruct((E, in_n), _F32),
                   jax.ShapeDtypeStruct((E, dm), _F32),
                   jax.ShapeDtypeStruct((8, dm), _F32)],
    )(ghs, ghd, e, wcx_t, wce_t, w1r_t, w1c_t, w1e_t,
      bc.reshape(1, -1), be1.reshape(1, -1))


def _sc_gather2(table, src2, dst2):
    n_rows, d = table.shape
    NW, n_ch, CH = src2.shape
    E = NW * n_ch * CH
    NSET = 4
    n_main = (n_ch - NSET) // NSET
    I32 = jnp.int32
    dt = table.dtype
    mesh = plsc.VectorSubcoreMesh(core_axis_name="c", subcore_axis_name="s")

    @functools.partial(
        pl.kernel, mesh=mesh,
        out_type=[jax.ShapeDtypeStruct((E, d), dt),
                  jax.ShapeDtypeStruct((E, d), dt)],
        scratch_types=(
            [pltpu.VMEM((n_ch, CH), I32), pltpu.VMEM((n_ch, CH), I32)]
            + [pltpu.VMEM((CH, d), dt) for _ in range(2 * NSET)]
            + [pltpu.SemaphoreType.DMA for _ in range(2 * NSET)]),
    )
    def k(tab_h, src_h, dst_h, os_h, od_h, *scr):
        idxs_v, idxd_v = scr[0], scr[1]
        bs = scr[2:2 + NSET]
        bd = scr[2 + NSET:2 + 2 * NSET]
        ss = scr[2 + 2 * NSET:2 + 3 * NSET]
        sd = scr[2 + 3 * NSET:2 + 4 * NSET]
        cid = lax.axis_index("c")
        sid = lax.axis_index("s")
        wid = sid * 2 + cid
        base_w = wid * (n_ch * CH)
        pltpu.sync_copy(src_h.at[wid], idxs_v)
        pltpu.sync_copy(dst_h.at[wid], idxd_v)

        def fire(k_, t):
            pltpu.async_copy(tab_h.at[idxs_v.at[k_]], bs[t], ss[t])
            pltpu.async_copy(tab_h.at[idxd_v.at[k_]], bd[t], sd[t])

        def drain_write(k_, t):
            dummy = tab_h.at[pl.ds(0, CH)]
            pltpu.make_async_copy(dummy, bs[t], ss[t]).wait()
            pltpu.make_async_copy(dummy, bd[t], sd[t]).wait()
            base = base_w + k_ * CH
            pltpu.sync_copy(bs[t], os_h.at[pl.ds(base, CH)])
            pltpu.sync_copy(bd[t], od_h.at[pl.ds(base, CH)])

        for t in range(NSET):
            fire(t, t)

        def grp(jj, carry):
            k0 = jj * NSET
            for t in range(NSET):
                drain_write(k0 + t, t)
                fire(k0 + t + NSET, t)
            return carry

        lax.fori_loop(0, n_main, grp, 0)
        base_k = NSET * n_main
        for t in range(NSET):
            if base_k + t < n_ch:
                drain_write(base_k + t, t)
        for k2 in range(base_k + NSET, n_ch):
            t = k2 % NSET
            fire(k2, t)
            drain_write(k2, t)

    return k(table, src2, dst2)


def _sc_scatter_add(c_arr, dst2, zeros_half):
    E, D = c_arr.shape
    n_nodes = zeros_half.shape[0]
    dh = D // 2
    n_tiles, n_ch, CH = dst2.shape
    per_t = n_ch * CH
    rows_a = (n_nodes // (n_tiles * 8)) * 8
    rem = n_nodes - rows_a * n_tiles
    NSET = 2
    n_main = (n_ch - NSET) // NSET
    mesh = plsc.VectorSubcoreMesh(core_axis_name="c", subcore_axis_name="s")

    @functools.partial(
        pl.kernel, mesh=mesh,
        out_type=jax.ShapeDtypeStruct((n_nodes, D), _F32),
        scratch_types=[pltpu.VMEM((n_ch, CH), jnp.int32),
                       pltpu.VMEM((CH, dh), _F32),
                       pltpu.VMEM((CH, dh), _F32),
                       pltpu.VMEM_SHARED((n_nodes, dh), _F32),
                       pltpu.SemaphoreType.DMA,
                       pltpu.SemaphoreType.DMA],
    )
    def k(c_h, dst_h, z_h, out_h, idx_v, buf0, buf1, acc, sem0, sem1):
        bufs = (buf0, buf1)
        sems = (sem0, sem1)
        cid = lax.axis_index("c")
        sid = lax.axis_index("s")
        r0 = sid * rows_a
        pltpu.sync_copy(z_h.at[pl.ds(r0, rows_a)], acc.at[pl.ds(r0, rows_a)])
        if rem:
            @pl.when(sid == 0)
            def _():
                pltpu.sync_copy(z_h.at[pl.ds(rows_a * n_tiles, rem)],
                                acc.at[pl.ds(rows_a * n_tiles, rem)])
        pltpu.sync_copy(dst_h.at[sid], idx_v)
        plsc.subcore_barrier()

        def fire(k_, t):
            base = sid * per_t + k_ * CH
            pltpu.async_copy(
                c_h.at[pl.ds(base, CH), pl.ds(cid * dh, dh)], bufs[t], sems[t])

        def drain_scatter(k_, t):
            pltpu.make_async_copy(
                c_h.at[pl.ds(0, CH), pl.ds(0, dh)], bufs[t], sems[t]).wait()
            pltpu.sync_copy(bufs[t], acc.at[idx_v.at[k_]], add=True)

        for t in range(NSET):
            fire(t, t)

        def grp(jj, carry):
            k0 = jj * NSET
            for t in range(NSET):
                drain_scatter(k0 + t, t)
                fire(k0 + t + NSET, t)
            return carry

        lax.fori_loop(0, n_main, grp, 0)
        base_k = NSET * n_main
        for t in range(NSET):
            if base_k + t < n_ch:
                drain_scatter(base_k + t, t)
        for k2 in range(base_k + NSET, n_ch):
            t = k2 % NSET
            fire(k2, t)
            drain_scatter(k2, t)

        plsc.subcore_barrier()
        pltpu.sync_copy(acc.at[pl.ds(r0, rows_a)],
                        out_h.at[pl.ds(r0, rows_a), pl.ds(cid * dh, dh)])
        if rem:
            @pl.when(sid == 0)
            def _():
                pltpu.sync_copy(
                    acc.at[pl.ds(rows_a * n_tiles, rem)],
                    out_h.at[pl.ds(rows_a * n_tiles, rem), pl.ds(cid * dh, dh)])

    return k(c_arr, dst2, zeros_half)


def kernel(x, edge_index, batch, edge_attr, params):
    n_nodes = x.shape[0]
    E = edge_index.shape[1]
    n_graphs = 128
    src = edge_index[0]
    dst = edge_index[1]
    src2 = src.reshape(32, -1, 40)
    dst2 = dst.reshape(32, -1, 40)
    dst2t = dst.reshape(16, -1, 80)
    zeros_half = jnp.zeros((n_nodes, 128), _F32)

    h = x
    e = edge_attr
    xs = []
    ghs, ghd = _sc_gather2(h, src2, dst2)
    for i, cv in enumerate(params['convs']):
        in_n = h.shape[1]
        in_e = e.shape[1]
        node_p = cv['node']
        edge_p = cv['edge']
        Wc = cv['Wc']
        wcx_t = Wc[:, :in_n].T
        wce_t = Wc[:, in_n:].T
        We1 = edge_p['W1']
        w1r_t = We1[:, :in_n].T
        w1c_t = We1[:, in_n:2 * in_n].T
        w1e_t = We1[:, 2 * in_n:].T
        c, z1, st1 = _edge_mms(ghs, ghd, e, wcx_t, wce_t, w1r_t, w1c_t,
                               w1e_t, cv['bc'], edge_p['b1'])
        agg = _sc_scatter_add(c, dst2t, zeros_half)
        e_mid, st_em = _mm([z1], [edge_p['W2'].T], edge_p['b2'],
                           bn=(st1, edge_p['g'], edge_p['bt'], float(E)),
                           stats=True)

        wn1_t = node_p['W1'].T
        zn, stn = _mm([h], [wn1_t], node_p['b1'],
                      eps=cv['eps'], pre_add=agg, stats=True)
        out_n, st_on = _mm([zn], [node_p['W2'].T], node_p['b2'],
                           bn=(stn, node_p['g'], node_p['bt'], float(n_nodes)),
                           stats=True)

        bn_i = params['bns'][i]
        h = _bn_relu(out_n, st_on, bn_i['g'], bn_i['b'], float(n_nodes))
        xs.append(h)
        if i + 1 < len(params['convs']):
            nghs, nghd = _sc_gather2(h, src2, dst2)
        e = _bn_relu(e_mid, st_em, bn_i['g'], bn_i['b'], float(E), relu=False)
        if i + 1 < len(params['convs']):
            ghs, ghd = nghs, nghd

    hcat = jnp.concatenate(xs, axis=1)
    pooled = _pool(hcat, batch, n_graphs)

    hid = xs[0].shape[1]
    emb = hcat.shape[1]
    wbd = jnp.zeros((emb, emb), _F32)
    bcat = jnp.concatenate([p['b'] for p in params['preds']])
    for i, p in enumerate(params['preds']):
        wbd = wbd.at[i * hid:(i + 1) * hid, i * hid:(i + 1) * hid].set(p['W'].T)
    xcat = _mm([pooled], [wbd], bcat, precision=None)

    graph_embedding = _ff_fused(xcat, params['global_d'])
    node_embedding = _ff_fused(hcat, params['local_d'])
    return (graph_embedding, node_embedding, xcat)


# speedup vs baseline: 1.0920x; 1.0920x over previous
import functools

import jax
import jax.numpy as jnp
from jax import lax
from jax.experimental import pallas as pl
from jax.experimental.pallas import tpu as pltpu
from jax.experimental.pallas import tpu_sc as plsc

_F32 = jnp.float32
_BN_EPS = 1e-5


def _pick_br(r):
    if r % 1000 == 0:
        return 1000
    return r


def _mm(xs, wts, bias=None, *, relu=False, adds=(), post_add=None,
        stats=False, bn=None, eps=None, pre_add=None, precision=None):
    xs = list(xs)
    wts = list(wts)
    adds = list(adds)
    R = xs[0].shape[0]
    N = wts[0].shape[1]
    BR = _pick_br(R)
    grid = R // BR
    nx = len(xs)
    na = len(adds)
    has_b = bias is not None
    has_pa = post_add is not None
    has_eps = eps is not None
    has_bn = bn is not None
    has_pre = pre_add is not None

    in_specs = []
    args = []
    for xa in xs:
        in_specs.append(pl.BlockSpec((BR, xa.shape[1]), lambda i: (i, 0)))
        args.append(xa)
    if has_pre:
        in_specs.append(pl.BlockSpec((BR, xs[0].shape[1]), lambda i: (i, 0)))
        args.append(pre_add)
    for wt in wts:
        in_specs.append(pl.BlockSpec(wt.shape, lambda i: (0, 0)))
        args.append(wt)
    if has_bn:
        st_in, g_in, bt_in, cnt = bn
        K0 = xs[0].shape[1]
        in_specs.append(pl.BlockSpec((8, K0), lambda i: (0, 0)))
        args.append(st_in)
        in_specs.append(pl.BlockSpec((1, K0), lambda i: (0, 0)))
        args.append(g_in.reshape(1, -1))
        in_specs.append(pl.BlockSpec((1, K0), lambda i: (0, 0)))
        args.append(bt_in.reshape(1, -1))
    if has_b:
        in_specs.append(pl.BlockSpec((1, N), lambda i: (0, 0)))
        args.append(bias.reshape(1, -1))
    for a in adds:
        in_specs.append(pl.BlockSpec((BR, N), lambda i: (i, 0)))
        args.append(a)
    if has_pa:
        in_specs.append(pl.BlockSpec((BR, N), lambda i: (i, 0)))
        args.append(post_add)
    if has_eps:
        in_specs.append(pl.BlockSpec((1, 1), lambda i: (0, 0)))
        args.append(jnp.reshape(eps, (1, 1)).astype(_F32))

    out_shape = [jax.ShapeDtypeStruct((R, N), _F32)]
    out_specs = [pl.BlockSpec((BR, N), lambda i: (i, 0))]
    if stats:
        out_shape.append(jax.ShapeDtypeStruct((8, N), _F32))
        out_specs.append(pl.BlockSpec((8, N), lambda i: (0, 0)))

    def body(*refs):
        p = 0
        xrefs = refs[p:p + nx]; p += nx
        if has_pre:
            pre_ref = refs[p]; p += 1
        wrefs = refs[p:p + nx]; p += nx
        if has_bn:
            st_ref, g_ref, bt_ref = refs[p:p + 3]; p += 3
        if has_b:
            b_ref = refs[p]; p += 1
        arefs = refs[p:p + na]; p += na
        if has_pa:
            pa_ref = refs[p]; p += 1
        if has_eps:
            eps_ref = refs[p]; p += 1
        out_ref = refs[p]; p += 1
        if stats:
            sto_ref = refs[p]; p += 1

        i = pl.program_id(0)
        if has_bn:
            stv = st_ref[...]
            s1 = stv[0:1, :]
            s2 = stv[1:2, :]
            mean = s1 / cnt
            var = s2 / cnt - mean * mean
            scale = g_ref[...] * lax.rsqrt(var + _BN_EPS)
            shift = bt_ref[...] - mean * scale
            x0 = jnp.maximum(xrefs[0][...] * scale + shift, 0.0)
        else:
            x0 = xrefs[0][...]
        if has_eps:
            x0 = x0 * (1.0 + eps_ref[0, 0])
        if has_pre:
            x0 = x0 + pre_ref[...]
        acc = jnp.dot(x0, wrefs[0][...], preferred_element_type=_F32,
                      precision=precision)
        for t in range(1, nx):
            acc = acc + jnp.dot(xrefs[t][...], wrefs[t][...],
                                preferred_element_type=_F32,
                                precision=precision)
        if has_b:
            acc = acc + b_ref[...]
        for ar in arefs:
            acc = acc + ar[...].astype(_F32)
        if relu:
            acc = jnp.maximum(acc, 0.0)
        if has_pa:
            acc = acc + pa_ref[...]
        out_ref[...] = acc
        if stats:
            c1 = jnp.sum(acc, axis=0, keepdims=True)
            c2 = jnp.sum(acc * acc, axis=0, keepdims=True)
            blk = jnp.concatenate(
                [c1, c2, jnp.zeros((6, N), _F32)], axis=0)

            @pl.when(i == 0)
            def _():
                sto_ref[...] = blk

            @pl.when(i != 0)
            def _():
                sto_ref[...] = sto_ref[...] + blk

    res = pl.pallas_call(
        body, grid=(grid,), in_specs=in_specs, out_specs=out_specs,
        out_shape=out_shape)(*args)
    if stats:
        return res[0], res[1]
    return res[0]


def _bn_relu(z, st, g, bt, cnt, relu=True):
    R, K = z.shape
    BR = _pick_br(R)
    grid = R // BR

    def body(z_ref, st_ref, g_ref, bt_ref, o_ref):
        stv = st_ref[...]
        mean = stv[0:1, :] / cnt
        var = stv[1:2, :] / cnt - mean * mean
        scale = g_ref[...] * lax.rsqrt(var + _BN_EPS)
        shift = bt_ref[...] - mean * scale
        o = z_ref[...] * scale + shift
        if relu:
            o = jnp.maximum(o, 0.0)
        o_ref[...] = o

    return pl.pallas_call(
        body, grid=(grid,),
        in_specs=[pl.BlockSpec((BR, K), lambda i: (i, 0)),
                  pl.BlockSpec((8, K), lambda i: (0, 0)),
                  pl.BlockSpec((1, K), lambda i: (0, 0)),
                  pl.BlockSpec((1, K), lambda i: (0, 0))],
        out_specs=pl.BlockSpec((BR, K), lambda i: (i, 0)),
        out_shape=jax.ShapeDtypeStruct((R, K), _F32),
    )(z, st, g.reshape(1, -1), bt.reshape(1, -1))


def _pool(hcat, batch, n_graphs):
    R, D = hcat.shape
    BRn = 2000 if R % 2000 == 0 else R
    grid = R // BRn
    b3 = batch.reshape(grid, 1, BRn)

    def body(h_ref, b_ref, o_ref):
        i = pl.program_id(0)
        bv = b_ref[...].reshape(1, BRn)
        gid = lax.broadcasted_iota(jnp.int32, (n_graphs, BRn), 0)
        oh = (gid == bv).astype(_F32)
        acc = jnp.dot(oh, h_ref[...], preferred_element_type=_F32,
                      precision=lax.Precision.HIGHEST)

        @pl.when(i == 0)
        def _():
            o_ref[...] = acc

        @pl.when(i != 0)
        def _():
            o_ref[...] = o_ref[...] + acc

    return pl.pallas_call(
        body, grid=(grid,),
        in_specs=[pl.BlockSpec((BRn, D), lambda i: (i, 0)),
                  pl.BlockSpec((1, 1, BRn), lambda i: (i, 0, 0))],
        out_specs=pl.BlockSpec((n_graphs, D), lambda i: (0, 0)),
        out_shape=jax.ShapeDtypeStruct((n_graphs, D), _F32),
    )(hcat, b3)


def _ff_fused(x, p):
    R, D = x.shape
    BR = 1000 if R % 1000 == 0 else R
    grid = R // BR
    w1t, w2t, w3t, wsct = p['W1'].T, p['W2'].T, p['W3'].T, p['Wsc'].T
    b1 = p['b1'].reshape(1, -1)
    b2 = p['b2'].reshape(1, -1)
    b3 = p['b3'].reshape(1, -1)
    bsc = p['bsc'].reshape(1, -1)

    def body(x_ref, w1_ref, w2_ref, w3_ref, wsc_ref,
             b1_ref, b2_ref, b3_ref, bsc_ref, o_ref):
        hi = None
        xv = x_ref[...]
        h1 = jnp.maximum(jnp.dot(xv, w1_ref[...], preferred_element_type=_F32,
                                 precision=hi) + b1_ref[...], 0.0)
        h2 = jnp.maximum(jnp.dot(h1, w2_ref[...], preferred_element_type=_F32,
                                 precision=hi) + b2_ref[...], 0.0)
        h3 = jnp.maximum(jnp.dot(h2, w3_ref[...], preferred_element_type=_F32,
                                 precision=hi) + b3_ref[...], 0.0)
        sc = jnp.dot(xv, wsc_ref[...], preferred_element_type=_F32,
                     precision=hi) + bsc_ref[...]
        o_ref[...] = h3 + sc

    wspec = pl.BlockSpec((D, D), lambda i: (0, 0))
    bspec = pl.BlockSpec((1, D), lambda i: (0, 0))
    return pl.pallas_call(
        body, grid=(grid,),
        in_specs=[pl.BlockSpec((BR, D), lambda i: (i, 0)),
                  wspec, wspec, wspec, wspec, bspec, bspec, bspec, bspec],
        out_specs=pl.BlockSpec((BR, D), lambda i: (i, 0)),
        out_shape=jax.ShapeDtypeStruct((R, D), _F32),
    )(x, w1t, w2t, w3t, wsct, b1, b2, b3, bsc)


def _edge_mms(ghs, ghd, e, wcx_t, wce_t, w1r_t, w1c_t, w1e_t, bc, be1):
    E, in_n = ghs.shape
    in_e = e.shape[1]
    dm = w1r_t.shape[1]
    BR = _pick_br(E)
    grid = E // BR

    def body(ghs_ref, ghd_ref, e_ref, wcx_ref, wce_ref, w1r_ref, w1c_ref,
             w1e_ref, bc_ref, be1_ref, c_ref, z1_ref, st_ref):
        i = pl.program_id(0)
        ghs_v = ghs_ref[...]
        e_v = e_ref[...]
        c = jnp.dot(ghs_v, wcx_ref[...], preferred_element_type=_F32)
        c = c + jnp.dot(e_v, wce_ref[...], preferred_element_type=_F32)
        c_ref[...] = jnp.maximum(c + bc_ref[...], 0.0)
        z1 = jnp.dot(ghs_v, w1r_ref[...], preferred_element_type=_F32)
        z1 = z1 + jnp.dot(ghd_ref[...], w1c_ref[...], preferred_element_type=_F32)
        z1 = z1 + jnp.dot(e_v, w1e_ref[...], preferred_element_type=_F32)
        z1 = z1 + be1_ref[...]
        z1_ref[...] = z1
        c1 = jnp.sum(z1, axis=0, keepdims=True)
        c2 = jnp.sum(z1 * z1, axis=0, keepdims=True)
        blk = jnp.concatenate([c1, c2, jnp.zeros((6, dm), _F32)], axis=0)

        @pl.when(i == 0)
        def _():
            st_ref[...] = blk

        @pl.when(i != 0)
        def _():
            st_ref[...] = st_ref[...] + blk

    row = lambda d: pl.BlockSpec((BR, d), lambda i: (i, 0))
    full = lambda sh: pl.BlockSpec(sh, lambda i: (0, 0))
    return pl.pallas_call(
        body, grid=(grid,),
        in_specs=[row(in_n), row(in_n), row(in_e),
                  full(wcx_t.shape), full(wce_t.shape), full(w1r_t.shape),
                  full(w1c_t.shape), full(w1e_t.shape),
                  full((1, in_n)), full((1, dm))],
        out_specs=[row(in_n), row(dm), full((8, dm))],
        out_shape=[jax.ShapeDtypeStruct((E, in_n), _F32),
                   jax.ShapeDtypeStruct((E, dm), _F32),
                   jax.ShapeDtypeStruct((8, dm), _F32)],
    )(ghs, ghd, e, wcx_t, wce_t, w1r_t, w1c_t, w1e_t,
      bc.reshape(1, -1), be1.reshape(1, -1))


def _sc_gather2(table, src2, dst2, need_dst=True):
    n_rows, d = table.shape
    NW, n_ch, CH = src2.shape
    E = NW * n_ch * CH
    NSET = 4
    n_main = (n_ch - NSET) // NSET
    I32 = jnp.int32
    dt = table.dtype
    mesh = plsc.VectorSubcoreMesh(core_axis_name="c", subcore_axis_name="s")

    @functools.partial(
        pl.kernel, mesh=mesh,
        out_type=[jax.ShapeDtypeStruct((E, d), dt),
                  jax.ShapeDtypeStruct((E, d), dt)],
        scratch_types=(
            [pltpu.VMEM((n_ch, CH), I32), pltpu.VMEM((n_ch, CH), I32)]
            + [pltpu.VMEM((CH, d), dt) for _ in range(2 * NSET)]
            + [pltpu.SemaphoreType.DMA for _ in range(2 * NSET)]),
    )
    def k(tab_h, src_h, dst_h, os_h, od_h, *scr):
        idxs_v, idxd_v = scr[0], scr[1]
        bs = scr[2:2 + NSET]
        bd = scr[2 + NSET:2 + 2 * NSET]
        ss = scr[2 + 2 * NSET:2 + 3 * NSET]
        sd = scr[2 + 3 * NSET:2 + 4 * NSET]
        cid = lax.axis_index("c")
        sid = lax.axis_index("s")
        wid = sid * 2 + cid
        base_w = wid * (n_ch * CH)
        pltpu.sync_copy(src_h.at[wid], idxs_v)
        if need_dst:
            pltpu.sync_copy(dst_h.at[wid], idxd_v)

        def fire(k_, t):
            pltpu.async_copy(tab_h.at[idxs_v.at[k_]], bs[t], ss[t])
            if need_dst:
                pltpu.async_copy(tab_h.at[idxd_v.at[k_]], bd[t], sd[t])

        def drain_write(k_, t):
            dummy = tab_h.at[pl.ds(0, CH)]
            pltpu.make_async_copy(dummy, bs[t], ss[t]).wait()
            base = base_w + k_ * CH
            pltpu.sync_copy(bs[t], os_h.at[pl.ds(base, CH)])
            if need_dst:
                pltpu.make_async_copy(dummy, bd[t], sd[t]).wait()
                pltpu.sync_copy(bd[t], od_h.at[pl.ds(base, CH)])

        for t in range(NSET):
            fire(t, t)

        def grp(jj, carry):
            k0 = jj * NSET
            for t in range(NSET):
                drain_write(k0 + t, t)
                fire(k0 + t + NSET, t)
            return carry

        lax.fori_loop(0, n_main, grp, 0)
        base_k = NSET * n_main
        for t in range(NSET):
            if base_k + t < n_ch:
                drain_write(base_k + t, t)
        for k2 in range(base_k + NSET, n_ch):
            t = k2 % NSET
            fire(k2, t)
            drain_write(k2, t)

    return k(table, src2, dst2)


def _sc_scatter_add(c_arr, dst2, zeros_half):
    E, D = c_arr.shape
    n_nodes = zeros_half.shape[0]
    dh = D // 2
    n_tiles, n_ch, CH = dst2.shape
    per_t = n_ch * CH
    rows_a = (n_nodes // (n_tiles * 8)) * 8
    rem = n_nodes - rows_a * n_tiles
    NSET = 2
    n_main = (n_ch - NSET) // NSET
    mesh = plsc.VectorSubcoreMesh(core_axis_name="c", subcore_axis_name="s")

    @functools.partial(
        pl.kernel, mesh=mesh,
        out_type=jax.ShapeDtypeStruct((n_nodes, D), _F32),
        scratch_types=[pltpu.VMEM((n_ch, CH), jnp.int32),
                       pltpu.VMEM((CH, dh), _F32),
                       pltpu.VMEM((CH, dh), _F32),
                       pltpu.VMEM_SHARED((n_nodes, dh), _F32),
                       pltpu.SemaphoreType.DMA,
                       pltpu.SemaphoreType.DMA],
    )
    def k(c_h, dst_h, z_h, out_h, idx_v, buf0, buf1, acc, sem0, sem1):
        bufs = (buf0, buf1)
        sems = (sem0, sem1)
        cid = lax.axis_index("c")
        sid = lax.axis_index("s")
        r0 = sid * rows_a
        pltpu.sync_copy(z_h.at[pl.ds(r0, rows_a)], acc.at[pl.ds(r0, rows_a)])
        if rem:
            @pl.when(sid == 0)
            def _():
                pltpu.sync_copy(z_h.at[pl.ds(rows_a * n_tiles, rem)],
                                acc.at[pl.ds(rows_a * n_tiles, rem)])
        pltpu.sync_copy(dst_h.at[sid], idx_v)
        plsc.subcore_barrier()

        def fire(k_, t):
            base = sid * per_t + k_ * CH
            pltpu.async_copy(
                c_h.at[pl.ds(base, CH), pl.ds(cid * dh, dh)], bufs[t], sems[t])

        def drain_scatter(k_, t):
            pltpu.make_async_copy(
                c_h.at[pl.ds(0, CH), pl.ds(0, dh)], bufs[t], sems[t]).wait()
            pltpu.sync_copy(bufs[t], acc.at[idx_v.at[k_]], add=True)

        for t in range(NSET):
            fire(t, t)

        def grp(jj, carry):
            k0 = jj * NSET
            for t in range(NSET):
                drain_scatter(k0 + t, t)
                fire(k0 + t + NSET, t)
            return carry

        lax.fori_loop(0, n_main, grp, 0)
        base_k = NSET * n_main
        for t in range(NSET):
            if base_k + t < n_ch:
                drain_scatter(base_k + t, t)
        for k2 in range(base_k + NSET, n_ch):
            t = k2 % NSET
            fire(k2, t)
            drain_scatter(k2, t)

        plsc.subcore_barrier()
        pltpu.sync_copy(acc.at[pl.ds(r0, rows_a)],
                        out_h.at[pl.ds(r0, rows_a), pl.ds(cid * dh, dh)])
        if rem:
            @pl.when(sid == 0)
            def _():
                pltpu.sync_copy(
                    acc.at[pl.ds(rows_a * n_tiles, rem)],
                    out_h.at[pl.ds(rows_a * n_tiles, rem), pl.ds(cid * dh, dh)])

    return k(c_arr, dst2, zeros_half)


def kernel(x, edge_index, batch, edge_attr, params):
    n_nodes = x.shape[0]
    E = edge_index.shape[1]
    n_graphs = 128
    src = edge_index[0]
    dst = edge_index[1]
    src2 = src.reshape(32, -1, 40)
    dst2 = dst.reshape(32, -1, 40)
    dst2t = dst.reshape(16, -1, 80)
    zeros_half = jnp.zeros((n_nodes, 128), _F32)

    h = x
    e = edge_attr
    xs = []
    ghs, ghd = _sc_gather2(h, src2, dst2)
    for i, cv in enumerate(params['convs']):
        in_n = h.shape[1]
        in_e = e.shape[1]
        node_p = cv['node']
        edge_p = cv['edge']
        Wc = cv['Wc']
        wcx_t = Wc[:, :in_n].T
        wce_t = Wc[:, in_n:].T
        We1 = edge_p['W1']
        w1r_t = We1[:, :in_n].T
        w1c_t = We1[:, in_n:2 * in_n].T
        w1e_t = We1[:, 2 * in_n:].T
        last = i + 1 == len(params['convs'])
        if last:
            c = _mm([ghs, e], [wcx_t, wce_t], cv['bc'], relu=True)
        else:
            c, z1, st1 = _edge_mms(ghs, ghd, e, wcx_t, wce_t, w1r_t, w1c_t,
                                   w1e_t, cv['bc'], edge_p['b1'])
        agg = _sc_scatter_add(c, dst2t, zeros_half)
        if not last:
            e_mid, st_em = _mm([z1], [edge_p['W2'].T], edge_p['b2'],
                               bn=(st1, edge_p['g'], edge_p['bt'], float(E)),
                               stats=True)

        wn1_t = node_p['W1'].T
        zn, stn = _mm([h], [wn1_t], node_p['b1'],
                      eps=cv['eps'], pre_add=agg, stats=True)
        out_n, st_on = _mm([zn], [node_p['W2'].T], node_p['b2'],
                           bn=(stn, node_p['g'], node_p['bt'], float(n_nodes)),
                           stats=True)

        bn_i = params['bns'][i]
        h = _bn_relu(out_n, st_on, bn_i['g'], bn_i['b'], float(n_nodes))
        xs.append(h)
        if not last:
            last_next = i + 2 == len(params['convs'])
            ghs, ghd = _sc_gather2(h, src2, dst2, need_dst=not last_next)
            e = _bn_relu(e_mid, st_em, bn_i['g'], bn_i['b'], float(E),
                         relu=False)

    hcat = jnp.concatenate(xs, axis=1)
    pooled = _pool(hcat, batch, n_graphs)

    hid = xs[0].shape[1]
    emb = hcat.shape[1]
    wbd = jnp.zeros((emb, emb), _F32)
    bcat = jnp.concatenate([p['b'] for p in params['preds']])
    for i, p in enumerate(params['preds']):
        wbd = wbd.at[i * hid:(i + 1) * hid, i * hid:(i + 1) * hid].set(p['W'].T)
    xcat = _mm([pooled], [wbd], bcat, precision=None)

    graph_embedding = _ff_fused(xcat, params['global_d'])
    node_embedding = _ff_fused(hcat, params['local_d'])
    return (graph_embedding, node_embedding, xcat)
